# baseline (speedup 1.0000x reference)
import functools

import jax
import jax.numpy as jnp
from jax import lax
from jax.experimental import pallas as pl
from jax.experimental.pallas import tpu as pltpu
from jax.experimental.pallas import tpu_sc as plsc

_N, _K, _D = 10000, 32, 128
_NW = 32
_NPW = 320
_NPAD = _NW * _NPW
_NC = 4
_CH = _NPW // _NC


_RPT = 632
_RLAST = _N - 15 * _RPT
_CHT = 128
_NCA = 5
_NCB = 3
_MA = 14
_MB = 3
_FASTC = 0


def _neighbor_sum_body(x_hbm, adjt_hbm, out_hbm, idx_v, acc_v, xs,
                       sem0, sem1, sem2):
    s = lax.axis_index("s")
    c = lax.axis_index("c")

    @pl.when(s < 15)
    def _():
        pltpu.async_copy(
            x_hbm.at[pl.ds(s * _RPT, _RPT)], xs.at[pl.ds(s * _RPT, _RPT)],
            sem0)

    @pl.when(s == 15)
    def _():
        pltpu.async_copy(
            x_hbm.at[pl.ds(15 * _RPT, _RLAST)],
            xs.at[pl.ds(15 * _RPT, _RLAST)], sem0)

    @pl.when(c == _FASTC)
    def _():
        pltpu.sync_copy(
            adjt_hbm.at[pl.ds(s * (_NCA + _NCB), _NCA)],
            idx_v.at[pl.ds(0, _NCA)])

    @pl.when(c != _FASTC)
    def _():
        pltpu.sync_copy(
            adjt_hbm.at[pl.ds(s * (_NCA + _NCB) + _NCA, _NCB)],
            idx_v.at[pl.ds(0, _NCB)])

    @pl.when(s < 15)
    def _():
        pltpu.make_async_copy(
            x_hbm.at[pl.ds(s * _RPT, _RPT)], xs.at[pl.ds(s * _RPT, _RPT)],
            sem0).wait()

    @pl.when(s == 15)
    def _():
        pltpu.make_async_copy(
            x_hbm.at[pl.ds(15 * _RPT, _RLAST)],
            xs.at[pl.ds(15 * _RPT, _RLAST)], sem0).wait()

    plsc.subcore_barrier()

    def _emit(cc, nc, m):
        ch0 = s * (_NCA + _NCB) + cc * _NCA
        passes = []
        done = 0
        while done < nc:
            take = min(2, nc - done)
            passes.append((done, take))
            done += take
        for p0, take in passes:
            for j in range(take):
                pltpu.async_copy(
                    xs.at[idx_v.at[p0 + j, 0]],
                    acc_v.at[pl.ds(j * 80, 80)], sem0)
            for j in range(take):
                pltpu.make_async_copy(
                    xs.at[idx_v.at[p0 + j, 0]],
                    acc_v.at[pl.ds(j * 80, 80)], sem0).wait()

            @pl.loop(1, m + 1)
            def _fh(k, _p0=p0, _take=take):
                for j in range(_take):
                    pltpu.async_copy(
                        x_hbm.at[idx_v.at[_p0 + j, k]],
                        acc_v.at[pl.ds(j * 80, 80)], sem2, add=True)

            @pl.loop(m + 1, _K)
            def _fs(k, _p0=p0, _take=take):
                for j in range(_take):
                    pltpu.async_copy(
                        xs.at[idx_v.at[_p0 + j, k]],
                        acc_v.at[pl.ds(j * 80, 80)], sem1, add=True)

            @pl.loop(1, m + 1)
            def _dh(k, _p0=p0, _take=take):
                for j in range(_take):
                    pltpu.make_async_copy(
                        x_hbm.at[idx_v.at[_p0 + j, k]],
                        acc_v.at[pl.ds(j * 80, 80)], sem2).wait()

            @pl.loop(m + 1, _K)
            def _ds(k, _p0=p0, _take=take):
                for j in range(_take):
                    pltpu.make_async_copy(
                        xs.at[idx_v.at[_p0 + j, k]],
                        acc_v.at[pl.ds(j * 80, 80)], sem1).wait()

            @pl.when(jnp.logical_or(s < 15, c == _FASTC))
            def _(_p0=p0, _take=take, _ch0=ch0):
                pltpu.sync_copy(
                    acc_v.at[pl.ds(0, _take * 80)],
                    out_hbm.at[pl.ds((_ch0 + _p0) * 80, _take * 80)])

    @pl.when(c == _FASTC)
    def _():
        _emit(0, _NCA, _MA)

    @pl.when(c != _FASTC)
    def _():
        _emit(1, _NCB, _MB)


def _make_neighbor_sum(interpret=False):
    mesh = plsc.VectorSubcoreMesh(
        core_axis_name="c", subcore_axis_name="s", num_cores=2,
        num_subcores=16)
    return functools.partial(
        pl.kernel,
        out_type=jax.ShapeDtypeStruct((_N, _D), jnp.float32),
        mesh=mesh,
        scratch_types=[
            pltpu.VMEM((_NCA, _K, 80), jnp.int32),
            pltpu.VMEM((160, _D), jnp.float32),
            pltpu.VMEM_SHARED((_N, _D), jnp.float32),
            pltpu.SemaphoreType.DMA,
            pltpu.SemaphoreType.DMA,
            pltpu.SemaphoreType.DMA,
        ],
        interpret=interpret,
    )(_neighbor_sum_body)


def _neighbor_sum(x, adjt):
    return _make_neighbor_sum()(x, adjt)


_BT = 400


def _bn_scale(g, va):
    return g * jax.lax.rsqrt(va + 1e-3)


def _gelu(y):
    return 0.5 * y * (1.0 + lax.erf(y * 0.7071067811865476))


def _ffn_x_body(x_ref, g_ref, be_ref, mu_ref, va_ref, w_ref, b_ref, out_ref):
    s = _bn_scale(g_ref[...], va_ref[...])
    t = be_ref[...] - mu_ref[...] * s
    y1 = jnp.dot(x_ref[...] * s + t, w_ref[...],
                 preferred_element_type=jnp.float32) + b_ref[...]
    out_ref[0] = _gelu(y1)


def _ffn_agg_body(buf_ref, x_ref, agg_ref, g_ref, be_ref, mu_ref, va_ref,
                  w_ref, b_ref, out_ref):
    del buf_ref
    s = _bn_scale(g_ref[...], va_ref[...])
    t = be_ref[...] - mu_ref[...] * s
    y2 = jnp.dot((x_ref[...] + agg_ref[...]) * s + t, w_ref[...],
                 preferred_element_type=jnp.float32) + b_ref[...]
    out_ref[0] = _gelu(y2)


def _vec_spec():
    return pl.BlockSpec((1, _D), lambda i: (0, 0))


def _ffn_x(x, g, be, mu, va, w, b):
    return pl.pallas_call(
        _ffn_x_body,
        grid=(_N // _BT,),
        in_specs=[
            pl.BlockSpec((_BT, _D), lambda i: (i, 0)),
            _vec_spec(), _vec_spec(), _vec_spec(), _vec_spec(),
            pl.BlockSpec((_D, _D), lambda i: (0, 0)),
            _vec_spec(),
        ],
        out_specs=pl.BlockSpec((1, _BT, _D), lambda i: (0, i, 0)),
        out_shape=jax.ShapeDtypeStruct((2, _N, _D), jnp.float32),
    )(x, g.reshape(1, _D), be.reshape(1, _D), mu.reshape(1, _D),
      va.reshape(1, _D), w, b.reshape(1, _D))


def _ffn_agg(buf, x, agg, g, be, mu, va, w, b):
    return pl.pallas_call(
        _ffn_agg_body,
        grid=(_N // _BT,),
        in_specs=[
            pl.BlockSpec(memory_space=pl.ANY),
            pl.BlockSpec((_BT, _D), lambda i: (i, 0)),
            pl.BlockSpec((_BT, _D), lambda i: (i, 0)),
            _vec_spec(), _vec_spec(), _vec_spec(), _vec_spec(),
            pl.BlockSpec((_D, _D), lambda i: (0, 0)),
            _vec_spec(),
        ],
        out_specs=pl.BlockSpec((1, _BT, _D), lambda i: (1, i, 0)),
        out_shape=jax.ShapeDtypeStruct((2, _N, _D), jnp.float32),
        input_output_aliases={0: 0},
    )(buf, x, agg, g.reshape(1, _D), be.reshape(1, _D), mu.reshape(1, _D),
      va.reshape(1, _D), w, b.reshape(1, _D))


def kernel(input_data, adj, edge_weights, bn_gamma, bn_beta, bn_mean, bn_var,
           W, b):
    x = input_data[0]
    adj_pad = jnp.concatenate(
        [adj.astype(jnp.int32), jnp.zeros((_CHT * 80 - _N, _K), jnp.int32)],
        axis=0)
    adjt = adj_pad.reshape(_CHT, 80, _K).transpose(0, 2, 1)
    nsum = _neighbor_sum(x, adjt)
    buf = _ffn_x(x, bn_gamma, bn_beta, bn_mean, bn_var, W, b)
    out2 = _ffn_agg(buf, x, nsum, bn_gamma, bn_beta, bn_mean, bn_var, W, b)
    return out2.reshape(1, 2 * _N, _D)


# speedup vs baseline: 1.1178x; 1.1178x over previous
import functools

import jax
import jax.numpy as jnp
from jax import lax
from jax.experimental import pallas as pl
from jax.experimental.pallas import tpu as pltpu
from jax.experimental.pallas import tpu_sc as plsc

_N, _K, _D = 10000, 32, 128
_NW = 32
_NPW = 320
_NPAD = _NW * _NPW
_NC = 4
_CH = _NPW // _NC


_RPT = 632
_RLAST = _N - 15 * _RPT
_NPP = 160
_NCP = _NPP // _CH
_LASTW = _NW - 1


def _neighbor_sum_body(x_hbm, adjt_hbm, out_hbm, idx_t, acc_v, xs,
                       sem0, sem1):
    s = lax.axis_index("s")
    wid = s * 2 + lax.axis_index("c")

    @pl.when(s < 15)
    def _():
        pltpu.async_copy(
            x_hbm.at[pl.ds(s * _RPT, _RPT)], xs.at[pl.ds(s * _RPT, _RPT)],
            sem0)

    @pl.when(s == 15)
    def _():
        pltpu.async_copy(
            x_hbm.at[pl.ds(15 * _RPT, _RLAST)],
            xs.at[pl.ds(15 * _RPT, _RLAST)], sem0)

    pltpu.sync_copy(adjt_hbm.at[wid], idx_t)

    @pl.when(s < 15)
    def _():
        pltpu.make_async_copy(
            x_hbm.at[pl.ds(s * _RPT, _RPT)], xs.at[pl.ds(s * _RPT, _RPT)],
            sem0).wait()

    @pl.when(s == 15)
    def _():
        pltpu.make_async_copy(
            x_hbm.at[pl.ds(15 * _RPT, _RLAST)],
            xs.at[pl.ds(15 * _RPT, _RLAST)], sem0).wait()

    plsc.subcore_barrier()

    for p in range(_NPW // _NPP):
        for c in range(_NCP):
            pltpu.async_copy(
                xs.at[idx_t.at[0, p * _NCP + c]],
                acc_v.at[pl.ds(c * _CH, _CH)], sem0)
        for c in range(_NCP):
            pltpu.make_async_copy(
                xs.at[idx_t.at[0, p * _NCP + c]],
                acc_v.at[pl.ds(c * _CH, _CH)], sem0).wait()

        @pl.loop(1, _K)
        def _fire(k, _p=p):
            for c in range(_NCP):
                pltpu.async_copy(
                    xs.at[idx_t.at[k, _p * _NCP + c]],
                    acc_v.at[pl.ds(c * _CH, _CH)], sem1, add=True)

        @pl.loop(1, _K)
        def _drain(k, _p=p):
            for c in range(_NCP):
                pltpu.make_async_copy(
                    xs.at[idx_t.at[k, _p * _NCP + c]],
                    acc_v.at[pl.ds(c * _CH, _CH)], sem1).wait()

        if p == 0:
            @pl.when(wid < _LASTW)
            def _():
                pltpu.sync_copy(
                    acc_v, out_hbm.at[pl.ds(wid * _NPW, _NPP)])

            @pl.when(wid == _LASTW)
            def _():
                pltpu.sync_copy(
                    acc_v.at[pl.ds(0, _NPP // 2)],
                    out_hbm.at[pl.ds(_LASTW * _NPW, _NPP // 2)])
        else:
            @pl.when(wid < _LASTW)
            def _():
                pltpu.sync_copy(
                    acc_v,
                    out_hbm.at[pl.ds(wid * _NPW + _NPP, _NPP)])


def _make_neighbor_sum(interpret=False):
    mesh = plsc.VectorSubcoreMesh(
        core_axis_name="c", subcore_axis_name="s", num_cores=2,
        num_subcores=16)
    return functools.partial(
        pl.kernel,
        out_type=jax.ShapeDtypeStruct((_N, _D), jnp.float32),
        mesh=mesh,
        scratch_types=[
            pltpu.VMEM((_K, _NC, _CH), jnp.int32),
            pltpu.VMEM((_NPP, _D), jnp.float32),
            pltpu.VMEM_SHARED((_N, _D), jnp.float32),
            pltpu.SemaphoreType.DMA,
            pltpu.SemaphoreType.DMA,
        ],
        interpret=interpret,
    )(_neighbor_sum_body)


def _neighbor_sum(x, adjt):
    return _make_neighbor_sum()(x, adjt)


_BT = 400


def _bn_scale(g, va):
    return g * jax.lax.rsqrt(va + 1e-3)


def _gelu(y):
    return 0.5 * y * (1.0 + lax.erf(y * 0.7071067811865476))


def _ffn_x_body(x_ref, g_ref, be_ref, mu_ref, va_ref, w_ref, b_ref, out_ref):
    s = _bn_scale(g_ref[...], va_ref[...])
    t = be_ref[...] - mu_ref[...] * s
    y1 = jnp.dot(x_ref[...] * s + t, w_ref[...],
                 preferred_element_type=jnp.float32) + b_ref[...]
    out_ref[0] = _gelu(y1)


def _ffn_agg_body(buf_ref, x_ref, agg_ref, g_ref, be_ref, mu_ref, va_ref,
                  w_ref, b_ref, out_ref):
    del buf_ref
    s = _bn_scale(g_ref[...], va_ref[...])
    t = be_ref[...] - mu_ref[...] * s
    y2 = jnp.dot((x_ref[...] + agg_ref[...]) * s + t, w_ref[...],
                 preferred_element_type=jnp.float32) + b_ref[...]
    out_ref[0] = _gelu(y2)


def _vec_spec():
    return pl.BlockSpec((1, _D), lambda i: (0, 0))


def _ffn_x(x, g, be, mu, va, w, b):
    return pl.pallas_call(
        _ffn_x_body,
        grid=(_N // _BT,),
        in_specs=[
            pl.BlockSpec((_BT, _D), lambda i: (i, 0)),
            _vec_spec(), _vec_spec(), _vec_spec(), _vec_spec(),
            pl.BlockSpec((_D, _D), lambda i: (0, 0)),
            _vec_spec(),
        ],
        out_specs=pl.BlockSpec((1, _BT, _D), lambda i: (0, i, 0)),
        out_shape=jax.ShapeDtypeStruct((2, _N, _D), jnp.float32),
    )(x, g.reshape(1, _D), be.reshape(1, _D), mu.reshape(1, _D),
      va.reshape(1, _D), w, b.reshape(1, _D))


def _ffn_agg(buf, x, agg, g, be, mu, va, w, b):
    return pl.pallas_call(
        _ffn_agg_body,
        grid=(_N // _BT,),
        in_specs=[
            pl.BlockSpec(memory_space=pl.ANY),
            pl.BlockSpec((_BT, _D), lambda i: (i, 0)),
            pl.BlockSpec((_BT, _D), lambda i: (i, 0)),
            _vec_spec(), _vec_spec(), _vec_spec(), _vec_spec(),
            pl.BlockSpec((_D, _D), lambda i: (0, 0)),
            _vec_spec(),
        ],
        out_specs=pl.BlockSpec((1, _BT, _D), lambda i: (1, i, 0)),
        out_shape=jax.ShapeDtypeStruct((2, _N, _D), jnp.float32),
        input_output_aliases={0: 0},
    )(buf, x, agg, g.reshape(1, _D), be.reshape(1, _D), mu.reshape(1, _D),
      va.reshape(1, _D), w, b.reshape(1, _D))


def kernel(input_data, adj, edge_weights, bn_gamma, bn_beta, bn_mean, bn_var,
           W, b):
    x = input_data[0]
    adj_pad = jnp.concatenate(
        [adj.astype(jnp.int32), jnp.zeros((_NPAD - _N, _K), jnp.int32)],
        axis=0)
    adjt = adj_pad.reshape(_NW, _NC, _CH, _K).transpose(0, 3, 1, 2)
    nsum = _neighbor_sum(x, adjt)
    buf = _ffn_x(x, bn_gamma, bn_beta, bn_mean, bn_var, W, b)
    out2 = _ffn_agg(buf, x, nsum, bn_gamma, bn_beta, bn_mean, bn_var, W, b)
    return out2.reshape(1, 2 * _N, _D)


# speedup vs baseline: 1.2041x; 1.0772x over previous
import functools

import jax
import jax.numpy as jnp
from jax import lax
from jax.experimental import pallas as pl
from jax.experimental.pallas import tpu as pltpu
from jax.experimental.pallas import tpu_sc as plsc

_N, _K, _D = 10000, 32, 128
_NW = 32
_NPW = 320
_NPAD = _NW * _NPW
_NC = 4
_CH = _NPW // _NC


_RPT = 632
_RLAST = _N - 15 * _RPT
_NPP = 160
_NCP = _NPP // _CH
_LASTW = _NW - 1


def _neighbor_sum_body(x_hbm, adjt_hbm, out_hbm, idx_t, acc_v, xs,
                       sem0, sem1):
    s = lax.axis_index("s")
    wid = s * 2 + lax.axis_index("c")

    @pl.when(s < 15)
    def _():
        pltpu.async_copy(
            x_hbm.at[pl.ds(s * _RPT, _RPT)], xs.at[pl.ds(s * _RPT, _RPT)],
            sem0)

    @pl.when(s == 15)
    def _():
        pltpu.async_copy(
            x_hbm.at[pl.ds(15 * _RPT, _RLAST)],
            xs.at[pl.ds(15 * _RPT, _RLAST)], sem0)

    pltpu.sync_copy(adjt_hbm.at[wid], idx_t)

    @pl.when(s < 15)
    def _():
        pltpu.make_async_copy(
            x_hbm.at[pl.ds(s * _RPT, _RPT)], xs.at[pl.ds(s * _RPT, _RPT)],
            sem0).wait()

    @pl.when(s == 15)
    def _():
        pltpu.make_async_copy(
            x_hbm.at[pl.ds(15 * _RPT, _RLAST)],
            xs.at[pl.ds(15 * _RPT, _RLAST)], sem0).wait()

    plsc.subcore_barrier()

    for p in range(_NPW // _NPP):
        for c in range(_NCP):
            pltpu.async_copy(
                xs.at[idx_t.at[0, p * _NCP + c]],
                acc_v.at[pl.ds(c * _CH, _CH)], sem0)
        for c in range(_NCP):
            pltpu.make_async_copy(
                xs.at[idx_t.at[0, p * _NCP + c]],
                acc_v.at[pl.ds(c * _CH, _CH)], sem0).wait()

        @pl.loop(1, _K)
        def _fire(k, _p=p):
            for c in range(_NCP):
                pltpu.async_copy(
                    xs.at[idx_t.at[k, _p * _NCP + c]],
                    acc_v.at[pl.ds(c * _CH, _CH)], sem1, add=True)

        @pl.loop(1, _K)
        def _drain(k, _p=p):
            for c in range(_NCP):
                pltpu.make_async_copy(
                    xs.at[idx_t.at[k, _p * _NCP + c]],
                    acc_v.at[pl.ds(c * _CH, _CH)], sem1).wait()

        if p == 0:
            @pl.when(wid < _LASTW)
            def _():
                pltpu.sync_copy(
                    acc_v, out_hbm.at[pl.ds(wid * _NPW, _NPP)])

            @pl.when(wid == _LASTW)
            def _():
                pltpu.sync_copy(
                    acc_v.at[pl.ds(0, _NPP // 2)],
                    out_hbm.at[pl.ds(_LASTW * _NPW, _NPP // 2)])
        else:
            @pl.when(wid < _LASTW)
            def _():
                pltpu.sync_copy(
                    acc_v,
                    out_hbm.at[pl.ds(wid * _NPW + _NPP, _NPP)])


def _make_neighbor_sum(interpret=False):
    mesh = plsc.VectorSubcoreMesh(
        core_axis_name="c", subcore_axis_name="s", num_cores=2,
        num_subcores=16)
    return functools.partial(
        pl.kernel,
        out_type=jax.ShapeDtypeStruct((_N, _D), jnp.float32),
        mesh=mesh,
        scratch_types=[
            pltpu.VMEM((_K, _NC, _CH), jnp.int32),
            pltpu.VMEM((_NPP, _D), jnp.float32),
            pltpu.VMEM_SHARED((_N, _D), jnp.float32),
            pltpu.SemaphoreType.DMA,
            pltpu.SemaphoreType.DMA,
        ],
        interpret=interpret,
    )(_neighbor_sum_body)


def _neighbor_sum(x, adjt):
    return _make_neighbor_sum()(x, adjt)


_BT = 1000


def _bn_scale(g, va):
    return g * jax.lax.rsqrt(va + 1e-3)


def _gelu(y):
    return 0.5 * y * (1.0 + lax.erf(y * 0.7071067811865476))


def _ffn_x_body(x_ref, g_ref, be_ref, mu_ref, va_ref, w_ref, b_ref, out_ref):
    s = _bn_scale(g_ref[...], va_ref[...])
    t = be_ref[...] - mu_ref[...] * s
    y1 = jnp.dot(x_ref[...] * s + t, w_ref[...],
                 preferred_element_type=jnp.float32) + b_ref[...]
    out_ref[0] = _gelu(y1)


def _ffn_agg_body(buf_ref, x_ref, agg_ref, g_ref, be_ref, mu_ref, va_ref,
                  w_ref, b_ref, out_ref):
    del buf_ref
    s = _bn_scale(g_ref[...], va_ref[...])
    t = be_ref[...] - mu_ref[...] * s
    y2 = jnp.dot((x_ref[...] + agg_ref[...]) * s + t, w_ref[...],
                 preferred_element_type=jnp.float32) + b_ref[...]
    out_ref[0] = _gelu(y2)


def _vec_spec():
    return pl.BlockSpec((1, _D), lambda i: (0, 0))


def _ffn_x(x, g, be, mu, va, w, b):
    return pl.pallas_call(
        _ffn_x_body,
        grid=(_N // _BT,),
        in_specs=[
            pl.BlockSpec((_BT, _D), lambda i: (i, 0)),
            _vec_spec(), _vec_spec(), _vec_spec(), _vec_spec(),
            pl.BlockSpec((_D, _D), lambda i: (0, 0)),
            _vec_spec(),
        ],
        out_specs=pl.BlockSpec((1, _BT, _D), lambda i: (0, i, 0)),
        out_shape=jax.ShapeDtypeStruct((2, _N, _D), jnp.float32),
    )(x, g.reshape(1, _D), be.reshape(1, _D), mu.reshape(1, _D),
      va.reshape(1, _D), w, b.reshape(1, _D))


def _ffn_agg(buf, x, agg, g, be, mu, va, w, b):
    return pl.pallas_call(
        _ffn_agg_body,
        grid=(_N // _BT,),
        in_specs=[
            pl.BlockSpec(memory_space=pl.ANY),
            pl.BlockSpec((_BT, _D), lambda i: (i, 0)),
            pl.BlockSpec((_BT, _D), lambda i: (i, 0)),
            _vec_spec(), _vec_spec(), _vec_spec(), _vec_spec(),
            pl.BlockSpec((_D, _D), lambda i: (0, 0)),
            _vec_spec(),
        ],
        out_specs=pl.BlockSpec((1, _BT, _D), lambda i: (1, i, 0)),
        out_shape=jax.ShapeDtypeStruct((2, _N, _D), jnp.float32),
        input_output_aliases={0: 0},
    )(buf, x, agg, g.reshape(1, _D), be.reshape(1, _D), mu.reshape(1, _D),
      va.reshape(1, _D), w, b.reshape(1, _D))


def kernel(input_data, adj, edge_weights, bn_gamma, bn_beta, bn_mean, bn_var,
           W, b):
    x = input_data[0]
    adj_pad = jnp.concatenate(
        [adj.astype(jnp.int32), jnp.zeros((_NPAD - _N, _K), jnp.int32)],
        axis=0)
    adjt = adj_pad.reshape(_NW, _NC, _CH, _K).transpose(0, 3, 1, 2)
    nsum = _neighbor_sum(x, adjt)
    buf = _ffn_x(x, bn_gamma, bn_beta, bn_mean, bn_var, W, b)
    out2 = _ffn_agg(buf, x, nsum, bn_gamma, bn_beta, bn_mean, bn_var, W, b)
    return out2.reshape(1, 2 * _N, _D)


# speedup vs baseline: 1.2319x; 1.0231x over previous
import functools

import jax
import jax.numpy as jnp
from jax import lax
from jax.experimental import pallas as pl
from jax.experimental.pallas import tpu as pltpu
from jax.experimental.pallas import tpu_sc as plsc

_N, _K, _D = 10000, 32, 128
_NW = 32
_NPW = 320
_NPAD = _NW * _NPW
_NC = 4
_CH = _NPW // _NC


_RPT = 632
_RLAST = _N - 15 * _RPT
_NPP = 160
_NCP = _NPP // _CH
_LASTW = _NW - 1


def _neighbor_sum_body(x_hbm, adjt_hbm, out_hbm, idx_t, acc_v, xs,
                       sem0, sem1):
    s = lax.axis_index("s")
    wid = s * 2 + lax.axis_index("c")

    @pl.when(s < 15)
    def _():
        pltpu.async_copy(
            x_hbm.at[pl.ds(s * _RPT, _RPT)], xs.at[pl.ds(s * _RPT, _RPT)],
            sem0)

    @pl.when(s == 15)
    def _():
        pltpu.async_copy(
            x_hbm.at[pl.ds(15 * _RPT, _RLAST)],
            xs.at[pl.ds(15 * _RPT, _RLAST)], sem0)

    pltpu.sync_copy(adjt_hbm.at[wid], idx_t)

    @pl.when(s < 15)
    def _():
        pltpu.make_async_copy(
            x_hbm.at[pl.ds(s * _RPT, _RPT)], xs.at[pl.ds(s * _RPT, _RPT)],
            sem0).wait()

    @pl.when(s == 15)
    def _():
        pltpu.make_async_copy(
            x_hbm.at[pl.ds(15 * _RPT, _RLAST)],
            xs.at[pl.ds(15 * _RPT, _RLAST)], sem0).wait()

    plsc.subcore_barrier()

    for p in range(_NPW // _NPP):
        for c in range(_NCP):
            pltpu.async_copy(
                xs.at[idx_t.at[0, p * _NCP + c]],
                acc_v.at[pl.ds(c * _CH, _CH)], sem0)
        for c in range(_NCP):
            pltpu.make_async_copy(
                xs.at[idx_t.at[0, p * _NCP + c]],
                acc_v.at[pl.ds(c * _CH, _CH)], sem0).wait()

        @pl.loop(1, _K)
        def _fire(k, _p=p):
            for c in range(_NCP):
                pltpu.async_copy(
                    xs.at[idx_t.at[k, _p * _NCP + c]],
                    acc_v.at[pl.ds(c * _CH, _CH)], sem1, add=True)

        @pl.loop(1, _K)
        def _drain(k, _p=p):
            for c in range(_NCP):
                pltpu.make_async_copy(
                    xs.at[idx_t.at[k, _p * _NCP + c]],
                    acc_v.at[pl.ds(c * _CH, _CH)], sem1).wait()

        if p == 0:
            @pl.when(wid < _LASTW)
            def _():
                pltpu.sync_copy(
                    acc_v, out_hbm.at[pl.ds(wid * _NPW, _NPP)])

            @pl.when(wid == _LASTW)
            def _():
                pltpu.sync_copy(
                    acc_v.at[pl.ds(0, _NPP // 2)],
                    out_hbm.at[pl.ds(_LASTW * _NPW, _NPP // 2)])
        else:
            @pl.when(wid < _LASTW)
            def _():
                pltpu.sync_copy(
                    acc_v,
                    out_hbm.at[pl.ds(wid * _NPW + _NPP, _NPP)])


def _make_neighbor_sum(interpret=False):
    mesh = plsc.VectorSubcoreMesh(
        core_axis_name="c", subcore_axis_name="s", num_cores=2,
        num_subcores=16)
    return functools.partial(
        pl.kernel,
        out_type=jax.ShapeDtypeStruct((_N, _D), jnp.float32),
        mesh=mesh,
        scratch_types=[
            pltpu.VMEM((_K, _NC, _CH), jnp.int32),
            pltpu.VMEM((_NPP, _D), jnp.float32),
            pltpu.VMEM_SHARED((_N, _D), jnp.float32),
            pltpu.SemaphoreType.DMA,
            pltpu.SemaphoreType.DMA,
        ],
        interpret=interpret,
    )(_neighbor_sum_body)


def _neighbor_sum(x, adjt):
    return _make_neighbor_sum()(x, adjt)


_BT = 2000


def _bn_scale(g, va):
    return g * jax.lax.rsqrt(va + 1e-3)


def _gelu(y):
    return 0.5 * y * (1.0 + lax.erf(y * 0.7071067811865476))


def _ffn_x_body(x_ref, g_ref, be_ref, mu_ref, va_ref, w_ref, b_ref, out_ref):
    s = _bn_scale(g_ref[...], va_ref[...])
    t = be_ref[...] - mu_ref[...] * s
    y1 = jnp.dot(x_ref[...] * s + t, w_ref[...],
                 preferred_element_type=jnp.float32) + b_ref[...]
    out_ref[0] = _gelu(y1)


def _ffn_agg_body(buf_ref, x_ref, agg_ref, g_ref, be_ref, mu_ref, va_ref,
                  w_ref, b_ref, out_ref):
    del buf_ref
    s = _bn_scale(g_ref[...], va_ref[...])
    t = be_ref[...] - mu_ref[...] * s
    y2 = jnp.dot((x_ref[...] + agg_ref[...]) * s + t, w_ref[...],
                 preferred_element_type=jnp.float32) + b_ref[...]
    out_ref[0] = _gelu(y2)


def _vec_spec():
    return pl.BlockSpec((1, _D), lambda i: (0, 0))


def _ffn_x(x, g, be, mu, va, w, b):
    return pl.pallas_call(
        _ffn_x_body,
        grid=(_N // _BT,),
        in_specs=[
            pl.BlockSpec((_BT, _D), lambda i: (i, 0)),
            _vec_spec(), _vec_spec(), _vec_spec(), _vec_spec(),
            pl.BlockSpec((_D, _D), lambda i: (0, 0)),
            _vec_spec(),
        ],
        out_specs=pl.BlockSpec((1, _BT, _D), lambda i: (0, i, 0)),
        out_shape=jax.ShapeDtypeStruct((2, _N, _D), jnp.float32),
    )(x, g.reshape(1, _D), be.reshape(1, _D), mu.reshape(1, _D),
      va.reshape(1, _D), w, b.reshape(1, _D))


def _ffn_agg(buf, x, agg, g, be, mu, va, w, b):
    return pl.pallas_call(
        _ffn_agg_body,
        grid=(_N // _BT,),
        in_specs=[
            pl.BlockSpec(memory_space=pl.ANY),
            pl.BlockSpec((_BT, _D), lambda i: (i, 0)),
            pl.BlockSpec((_BT, _D), lambda i: (i, 0)),
            _vec_spec(), _vec_spec(), _vec_spec(), _vec_spec(),
            pl.BlockSpec((_D, _D), lambda i: (0, 0)),
            _vec_spec(),
        ],
        out_specs=pl.BlockSpec((1, _BT, _D), lambda i: (1, i, 0)),
        out_shape=jax.ShapeDtypeStruct((2, _N, _D), jnp.float32),
        input_output_aliases={0: 0},
    )(buf, x, agg, g.reshape(1, _D), be.reshape(1, _D), mu.reshape(1, _D),
      va.reshape(1, _D), w, b.reshape(1, _D))


def kernel(input_data, adj, edge_weights, bn_gamma, bn_beta, bn_mean, bn_var,
           W, b):
    x = input_data[0]
    adj_pad = jnp.concatenate(
        [adj.astype(jnp.int32), jnp.zeros((_NPAD - _N, _K), jnp.int32)],
        axis=0)
    adjt = adj_pad.reshape(_NW, _NC, _CH, _K).transpose(0, 3, 1, 2)
    nsum = _neighbor_sum(x, adjt)
    buf = _ffn_x(x, bn_gamma, bn_beta, bn_mean, bn_var, W, b)
    out2 = _ffn_agg(buf, x, nsum, bn_gamma, bn_beta, bn_mean, bn_var, W, b)
    return out2.reshape(1, 2 * _N, _D)


# speedup vs baseline: 1.2566x; 1.0201x over previous
import functools

import jax
import jax.numpy as jnp
from jax import lax
from jax.experimental import pallas as pl
from jax.experimental.pallas import tpu as pltpu
from jax.experimental.pallas import tpu_sc as plsc

_N, _K, _D = 10000, 32, 128
_NW = 32
_NPW = 320
_NPAD = _NW * _NPW
_NC = 4
_CH = _NPW // _NC


_RPT = 632
_RLAST = _N - 15 * _RPT
_NPP = 160
_NCP = _NPP // _CH
_LASTW = _NW - 1


def _neighbor_sum_body(x_hbm, adjt_hbm, out_hbm, idx_t, acc_v, xs,
                       sem0, sem1):
    s = lax.axis_index("s")
    wid = s * 2 + lax.axis_index("c")

    @pl.when(s < 15)
    def _():
        pltpu.async_copy(
            x_hbm.at[pl.ds(s * _RPT, _RPT)], xs.at[pl.ds(s * _RPT, _RPT)],
            sem0)

    @pl.when(s == 15)
    def _():
        pltpu.async_copy(
            x_hbm.at[pl.ds(15 * _RPT, _RLAST)],
            xs.at[pl.ds(15 * _RPT, _RLAST)], sem0)

    pltpu.sync_copy(adjt_hbm.at[wid], idx_t)

    @pl.when(s < 15)
    def _():
        pltpu.make_async_copy(
            x_hbm.at[pl.ds(s * _RPT, _RPT)], xs.at[pl.ds(s * _RPT, _RPT)],
            sem0).wait()

    @pl.when(s == 15)
    def _():
        pltpu.make_async_copy(
            x_hbm.at[pl.ds(15 * _RPT, _RLAST)],
            xs.at[pl.ds(15 * _RPT, _RLAST)], sem0).wait()

    plsc.subcore_barrier()

    for p in range(_NPW // _NPP):
        for c in range(_NCP):
            pltpu.async_copy(
                xs.at[idx_t.at[0, p * _NCP + c]],
                acc_v.at[pl.ds(c * _CH, _CH)], sem0)
        for c in range(_NCP):
            pltpu.make_async_copy(
                xs.at[idx_t.at[0, p * _NCP + c]],
                acc_v.at[pl.ds(c * _CH, _CH)], sem0).wait()

        @pl.loop(1, _K)
        def _fire(k, _p=p):
            for c in range(_NCP):
                pltpu.async_copy(
                    xs.at[idx_t.at[k, _p * _NCP + c]],
                    acc_v.at[pl.ds(c * _CH, _CH)], sem1, add=True)

        @pl.loop(1, _K)
        def _drain(k, _p=p):
            for c in range(_NCP):
                pltpu.make_async_copy(
                    xs.at[idx_t.at[k, _p * _NCP + c]],
                    acc_v.at[pl.ds(c * _CH, _CH)], sem1).wait()

        if p == 0:
            @pl.when(wid < _LASTW)
            def _():
                pltpu.sync_copy(
                    acc_v, out_hbm.at[pl.ds(wid * _NPW, _NPP)])

            @pl.when(wid == _LASTW)
            def _():
                pltpu.sync_copy(
                    acc_v.at[pl.ds(0, _NPP // 2)],
                    out_hbm.at[pl.ds(_LASTW * _NPW, _NPP // 2)])
        else:
            @pl.when(wid < _LASTW)
            def _():
                pltpu.sync_copy(
                    acc_v,
                    out_hbm.at[pl.ds(wid * _NPW + _NPP, _NPP)])


def _make_neighbor_sum(interpret=False):
    mesh = plsc.VectorSubcoreMesh(
        core_axis_name="c", subcore_axis_name="s", num_cores=2,
        num_subcores=16)
    return functools.partial(
        pl.kernel,
        out_type=jax.ShapeDtypeStruct((_N, _D), jnp.float32),
        mesh=mesh,
        scratch_types=[
            pltpu.VMEM((_K, _NC, _CH), jnp.int32),
            pltpu.VMEM((_NPP, _D), jnp.float32),
            pltpu.VMEM_SHARED((_N, _D), jnp.float32),
            pltpu.SemaphoreType.DMA,
            pltpu.SemaphoreType.DMA,
        ],
        interpret=interpret,
    )(_neighbor_sum_body)


def _neighbor_sum(x, adjt):
    return _make_neighbor_sum()(x, adjt)


_BT = 5000


def _bn_scale(g, va):
    return g * jax.lax.rsqrt(va + 1e-3)


def _gelu(y):
    return 0.5 * y * (1.0 + lax.erf(y * 0.7071067811865476))


def _ffn_x_body(x_ref, g_ref, be_ref, mu_ref, va_ref, w_ref, b_ref, out_ref):
    s = _bn_scale(g_ref[...], va_ref[...])
    t = be_ref[...] - mu_ref[...] * s
    y1 = jnp.dot(x_ref[...] * s + t, w_ref[...],
                 preferred_element_type=jnp.float32) + b_ref[...]
    out_ref[0] = _gelu(y1)


def _ffn_agg_body(buf_ref, x_ref, agg_ref, g_ref, be_ref, mu_ref, va_ref,
                  w_ref, b_ref, out_ref):
    del buf_ref
    s = _bn_scale(g_ref[...], va_ref[...])
    t = be_ref[...] - mu_ref[...] * s
    y2 = jnp.dot((x_ref[...] + agg_ref[...]) * s + t, w_ref[...],
                 preferred_element_type=jnp.float32) + b_ref[...]
    out_ref[0] = _gelu(y2)


def _vec_spec():
    return pl.BlockSpec((1, _D), lambda i: (0, 0))


def _ffn_x(x, g, be, mu, va, w, b):
    return pl.pallas_call(
        _ffn_x_body,
        grid=(_N // _BT,),
        in_specs=[
            pl.BlockSpec((_BT, _D), lambda i: (i, 0)),
            _vec_spec(), _vec_spec(), _vec_spec(), _vec_spec(),
            pl.BlockSpec((_D, _D), lambda i: (0, 0)),
            _vec_spec(),
        ],
        out_specs=pl.BlockSpec((1, _BT, _D), lambda i: (0, i, 0)),
        out_shape=jax.ShapeDtypeStruct((2, _N, _D), jnp.float32),
    )(x, g.reshape(1, _D), be.reshape(1, _D), mu.reshape(1, _D),
      va.reshape(1, _D), w, b.reshape(1, _D))


def _ffn_agg(buf, x, agg, g, be, mu, va, w, b):
    return pl.pallas_call(
        _ffn_agg_body,
        grid=(_N // _BT,),
        in_specs=[
            pl.BlockSpec(memory_space=pl.ANY),
            pl.BlockSpec((_BT, _D), lambda i: (i, 0)),
            pl.BlockSpec((_BT, _D), lambda i: (i, 0)),
            _vec_spec(), _vec_spec(), _vec_spec(), _vec_spec(),
            pl.BlockSpec((_D, _D), lambda i: (0, 0)),
            _vec_spec(),
        ],
        out_specs=pl.BlockSpec((1, _BT, _D), lambda i: (1, i, 0)),
        out_shape=jax.ShapeDtypeStruct((2, _N, _D), jnp.float32),
        input_output_aliases={0: 0},
    )(buf, x, agg, g.reshape(1, _D), be.reshape(1, _D), mu.reshape(1, _D),
      va.reshape(1, _D), w, b.reshape(1, _D))


def kernel(input_data, adj, edge_weights, bn_gamma, bn_beta, bn_mean, bn_var,
           W, b):
    x = input_data[0]
    adj_pad = jnp.concatenate(
        [adj.astype(jnp.int32), jnp.zeros((_NPAD - _N, _K), jnp.int32)],
        axis=0)
    adjt = adj_pad.reshape(_NW, _NC, _CH, _K).transpose(0, 3, 1, 2)
    nsum = _neighbor_sum(x, adjt)
    buf = _ffn_x(x, bn_gamma, bn_beta, bn_mean, bn_var, W, b)
    out2 = _ffn_agg(buf, x, nsum, bn_gamma, bn_beta, bn_mean, bn_var, W, b)
    return out2.reshape(1, 2 * _N, _D)
